# baseline (speedup 1.0000x reference)
import jax
import jax.numpy as jnp
from jax.experimental import pallas as pl


def _copy_block(x_ref, o_ref):
    o_ref[...] = x_ref[...]


def kernel(features, labels):
    del labels
    n, h, w = features.shape
    flat = features.reshape(n * h, w)
    rows = n * h
    block_rows = 8192
    out = pl.pallas_call(
        _copy_block,
        grid=(rows // block_rows,),
        in_specs=[pl.BlockSpec((block_rows, w), lambda i: (i, 0))],
        out_specs=pl.BlockSpec((block_rows, w), lambda i: (i, 0)),
        out_shape=jax.ShapeDtypeStruct((rows, w), features.dtype),
    )(flat)
    return out.reshape(n, h, w)
